# Initial kernel scaffold; baseline (speedup 1.0000x reference)
import jax
import jax.numpy as jnp
from jax.experimental import pallas as pl


def kernel(x):
    raise NotImplementedError("write your pallas kernel here")


# speedup vs baseline: 1.4012x; 1.4012x over previous
import functools

import jax
import jax.numpy as jnp
import numpy as np
from jax.experimental import pallas as pl

N_PCT = 10
MIN_PCT = 5
MAX_PCT = 95


def _pct_constants(n):
    fracs = np.linspace(MIN_PCT / 100.0, MAX_PCT / 100.0, N_PCT)
    idx_float = fracs * (n - 1)
    idx_lower = np.floor(idx_float).astype(np.int32)
    idx_upper = np.ceil(idx_float).astype(np.int32)
    w_upper = (idx_float - idx_lower).astype(np.float32)
    return idx_lower, idx_upper, w_upper


def _body(x_ref, o_ref, *, n):
    v = x_ref[0]
    logn = int(np.log2(n))
    row = jax.lax.broadcasted_iota(jnp.int32, (n, 1), 0)
    bit0 = [((row >> j) & 1) == 0 for j in range(logn + 1)]
    for k in range(1, logn + 1):
        for j in reversed(range(k)):
            dist = 1 << j
            up = jnp.roll(v, -dist, axis=0)
            down = jnp.roll(v, dist, axis=0)
            partner = jnp.where(bit0[j], up, down)
            lo = jnp.minimum(v, partner)
            hi = jnp.maximum(v, partner)
            take_lo = bit0[j] == bit0[k]
            v = jnp.where(take_lo, lo, hi)

    idx_lower, idx_upper, w_upper = _pct_constants(n)
    rows = []
    for p in range(N_PCT):
        vl = v[idx_lower[p] : idx_lower[p] + 1, :]
        vu = v[idx_upper[p] : idx_upper[p] + 1, :]
        w = float(w_upper[p])
        rows.append(vl * (1.0 - w) + vu * w)
    o_ref[0] = jnp.concatenate(rows, axis=0)


@jax.jit
def kernel(x):
    b, n, d = x.shape
    DC = 128
    out = pl.pallas_call(
        functools.partial(_body, n=n),
        grid=(b, d // DC),
        in_specs=[pl.BlockSpec((1, n, DC), lambda i, j: (i, 0, j))],
        out_specs=pl.BlockSpec((1, N_PCT, DC), lambda i, j: (i, 0, j)),
        out_shape=jax.ShapeDtypeStruct((b, N_PCT, d), jnp.float32),
    )(x)
    return jnp.transpose(out, (0, 2, 1)).reshape(b, d * N_PCT)


# speedup vs baseline: 1.5650x; 1.1169x over previous
import functools

import jax
import jax.numpy as jnp
import numpy as np
from jax.experimental import pallas as pl

N_PCT = 10
MIN_PCT = 5
MAX_PCT = 95


def _pct_constants(n):
    fracs = np.linspace(MIN_PCT / 100.0, MAX_PCT / 100.0, N_PCT)
    idx_float = fracs * (n - 1)
    idx_lower = np.floor(idx_float).astype(np.int32)
    idx_upper = np.ceil(idx_float).astype(np.int32)
    w_upper = (idx_float - idx_lower).astype(np.float32)
    return idx_lower, idx_upper, w_upper


def _body(x_ref, o_ref, *, n):
    dc = x_ref.shape[2]
    v = x_ref[0]
    logn = int(np.log2(n))
    row = jax.lax.broadcasted_iota(jnp.int32, (n, 1), 0)
    bit0 = [((row >> j) & 1) == 0 for j in range(logn + 1)]
    row3 = jax.lax.broadcasted_iota(jnp.int32, (n // 8, 8, 1), 1)
    bit0_3 = [((row3 >> j) & 1) == 0 for j in range(3)]
    for k in range(1, logn + 1):
        for j in reversed(range(k)):
            dist = 1 << j
            if j < 3:
                w = v.reshape(n // 8, 8, dc)
                up = jnp.roll(w, -dist, axis=1)
                down = jnp.roll(w, dist, axis=1)
                partner = jnp.where(bit0_3[j], up, down)
                lo = jnp.minimum(w, partner)
                hi = jnp.maximum(w, partner)
                take_lo = (bit0_3[j] == bit0[k].reshape(n // 8, 8, 1))
                v = jnp.where(take_lo, lo, hi).reshape(n, dc)
            else:
                up = jnp.roll(v, -dist, axis=0)
                down = jnp.roll(v, dist, axis=0)
                partner = jnp.where(bit0[j], up, down)
                lo = jnp.minimum(v, partner)
                hi = jnp.maximum(v, partner)
                take_lo = bit0[j] == bit0[k]
                v = jnp.where(take_lo, lo, hi)

    idx_lower, idx_upper, w_upper = _pct_constants(n)
    rows = []
    for p in range(N_PCT):
        vl = v[idx_lower[p] : idx_lower[p] + 1, :]
        vu = v[idx_upper[p] : idx_upper[p] + 1, :]
        w = float(w_upper[p])
        rows.append(vl * (1.0 - w) + vu * w)
    o_ref[0] = jnp.concatenate(rows, axis=0)


@jax.jit
def kernel(x):
    b, n, d = x.shape
    DC = 128
    out = pl.pallas_call(
        functools.partial(_body, n=n),
        grid=(b, d // DC),
        in_specs=[pl.BlockSpec((1, n, DC), lambda i, j: (i, 0, j))],
        out_specs=pl.BlockSpec((1, N_PCT, DC), lambda i, j: (i, 0, j)),
        out_shape=jax.ShapeDtypeStruct((b, N_PCT, d), jnp.float32),
    )(x)
    return jnp.transpose(out, (0, 2, 1)).reshape(b, d * N_PCT)


# speedup vs baseline: 2.5680x; 1.6409x over previous
import functools

import jax
import jax.numpy as jnp
import numpy as np
from jax.experimental import pallas as pl

N_PCT = 10
MIN_PCT = 5
MAX_PCT = 95


def _pct_constants(n):
    fracs = np.linspace(MIN_PCT / 100.0, MAX_PCT / 100.0, N_PCT)
    idx_float = fracs * (n - 1)
    idx_lower = np.floor(idx_float).astype(np.int32)
    idx_upper = np.ceil(idx_float).astype(np.int32)
    w_upper = (idx_float - idx_lower).astype(np.float32)
    return idx_lower, idx_upper, w_upper


def _body(x_ref, o_ref, *, n):
    dc = x_ref.shape[2]
    v = x_ref[0]
    logn = int(np.log2(n))
    row = jax.lax.broadcasted_iota(jnp.int32, (n, 1), 0)
    bit0 = [((row >> j) & 1) == 0 for j in range(logn + 1)]
    row3 = jax.lax.broadcasted_iota(jnp.int32, (n // 8, 8, 1), 1)
    bit0_3 = [((row3 >> j) & 1) == 0 for j in range(3)]
    G = 3
    for k in range(1, logn + 1):
        big = [j for j in range(k - 1, -1, -1) if j >= 3]
        small = [j for j in range(min(k - 1, 2), -1, -1)]
        pos = 0
        while pos < len(big):
            group = big[pos : pos + G]
            pos += len(group)
            g = len(group)
            j0 = group[-1]
            dist = 1 << j0
            m = n // ((1 << g) * dist)
            w = v.reshape(m, 1 << g, dist, dc)
            sl = [w[:, q] for q in range(1 << g)]
            for j in group:
                e = j - j0
                step = 1 << e
                if k == logn:
                    asc = None
                else:
                    im = jax.lax.broadcasted_iota(jnp.int32, (m, 1, 1), 0)
                    asc = ((im >> (k - j0 - g)) & 1) == 0
                for q in range(1 << g):
                    if q & step:
                        continue
                    a, b_ = sl[q], sl[q ^ step]
                    lo = jnp.minimum(a, b_)
                    hi = jnp.maximum(a, b_)
                    if asc is None:
                        sl[q], sl[q ^ step] = lo, hi
                    else:
                        sl[q] = jnp.where(asc, lo, hi)
                        sl[q ^ step] = jnp.where(asc, hi, lo)
            v = jnp.stack(sl, axis=1).reshape(n, dc)
        for j in small:
            dist = 1 << j
            w = v.reshape(n // 8, 8, dc)
            up = jnp.roll(w, -dist, axis=1)
            down = jnp.roll(w, dist, axis=1)
            partner = jnp.where(bit0_3[j], up, down)
            lo = jnp.minimum(w, partner)
            hi = jnp.maximum(w, partner)
            take_lo = (bit0_3[j] == bit0[k].reshape(n // 8, 8, 1))
            v = jnp.where(take_lo, lo, hi).reshape(n, dc)

    idx_lower, idx_upper, w_upper = _pct_constants(n)
    rows = []
    for p in range(N_PCT):
        vl = v[idx_lower[p] : idx_lower[p] + 1, :]
        vu = v[idx_upper[p] : idx_upper[p] + 1, :]
        w = float(w_upper[p])
        rows.append(vl * (1.0 - w) + vu * w)
    o_ref[0] = jnp.concatenate(rows, axis=0)


@jax.jit
def kernel(x):
    b, n, d = x.shape
    DC = 128
    out = pl.pallas_call(
        functools.partial(_body, n=n),
        grid=(b, d // DC),
        in_specs=[pl.BlockSpec((1, n, DC), lambda i, j: (i, 0, j))],
        out_specs=pl.BlockSpec((1, N_PCT, DC), lambda i, j: (i, 0, j)),
        out_shape=jax.ShapeDtypeStruct((b, N_PCT, d), jnp.float32),
    )(x)
    return jnp.transpose(out, (0, 2, 1)).reshape(b, d * N_PCT)


# speedup vs baseline: 5.0401x; 1.9627x over previous
import functools

import jax
import jax.numpy as jnp
import numpy as np
from jax.experimental import pallas as pl

N_PCT = 10
MIN_PCT = 5
MAX_PCT = 95


def _pct_constants(n):
    fracs = np.linspace(MIN_PCT / 100.0, MAX_PCT / 100.0, N_PCT)
    idx_float = fracs * (n - 1)
    idx_lower = np.floor(idx_float).astype(np.int32)
    idx_upper = np.ceil(idx_float).astype(np.int32)
    w_upper = (idx_float - idx_lower).astype(np.float32)
    return idx_lower, idx_upper, w_upper


def _bitrev(r, nbits):
    return int(format(r, "0{}b".format(nbits))[::-1], 2)


def _body(x_ref, o_ref, *, n):
    dc = x_ref.shape[2]
    v = x_ref[0]
    logn = int(np.log2(n))
    row3 = jax.lax.broadcasted_iota(jnp.int32, (n // 8, 8, 1), 1)
    bit0_3 = [((row3 >> p) & 1) == 0 for p in range(3)]
    G = 3
    for k in range(1, logn + 1):
        ps = list(range(logn - k, logn))
        db = logn - 1 - k
        for p in [q for q in ps if q < 3]:
            dist = 1 << p
            w = v.reshape(n // 8, 8, dc)
            up = jnp.roll(w, -dist, axis=1)
            down = jnp.roll(w, dist, axis=1)
            partner = jnp.where(bit0_3[p], up, down)
            lo = jnp.minimum(w, partner)
            hi = jnp.maximum(w, partner)
            if k == logn:
                take_lo = bit0_3[p]
            else:
                take_lo = bit0_3[p] == bit0_3[db]
            v = jnp.where(take_lo, lo, hi).reshape(n, dc)
        big = [q for q in ps if q >= 3]
        pos = 0
        while pos < len(big):
            group = big[pos : pos + G]
            pos += len(group)
            g = len(group)
            p0 = group[0]
            dist = 1 << p0
            m = n // ((1 << g) * dist)
            w = v.reshape(m, 1 << g, dist, dc)
            sl = [w[:, q] for q in range(1 << g)]
            if k == logn:
                asc = None
            else:
                it = jax.lax.broadcasted_iota(jnp.int32, (1, dist, 1), 1)
                asc = ((it >> db) & 1) == 0
            for e in range(g):
                step = 1 << e
                for q in range(1 << g):
                    if q & step:
                        continue
                    a, b_ = sl[q], sl[q ^ step]
                    lo = jnp.minimum(a, b_)
                    hi = jnp.maximum(a, b_)
                    if asc is None:
                        sl[q], sl[q ^ step] = lo, hi
                    else:
                        sl[q] = jnp.where(asc, lo, hi)
                        sl[q ^ step] = jnp.where(asc, hi, lo)
            v = jnp.stack(sl, axis=1).reshape(n, dc)

    idx_lower, idx_upper, w_upper = _pct_constants(n)
    rows = []
    for p in range(N_PCT):
        rl = _bitrev(int(idx_lower[p]), logn)
        ru = _bitrev(int(idx_upper[p]), logn)
        vl = v[rl : rl + 1, :]
        vu = v[ru : ru + 1, :]
        w = float(w_upper[p])
        rows.append(vl * (1.0 - w) + vu * w)
    o_ref[0] = jnp.concatenate(rows, axis=0)


@jax.jit
def kernel(x):
    b, n, d = x.shape
    DC = 128
    out = pl.pallas_call(
        functools.partial(_body, n=n),
        grid=(b, d // DC),
        in_specs=[pl.BlockSpec((1, n, DC), lambda i, j: (i, 0, j))],
        out_specs=pl.BlockSpec((1, N_PCT, DC), lambda i, j: (i, 0, j)),
        out_shape=jax.ShapeDtypeStruct((b, N_PCT, d), jnp.float32),
    )(x)
    return jnp.transpose(out, (0, 2, 1)).reshape(b, d * N_PCT)


# speedup vs baseline: 5.6419x; 1.1194x over previous
import functools

import jax
import jax.numpy as jnp
import numpy as np
from jax.experimental import pallas as pl

N_PCT = 10
MIN_PCT = 5
MAX_PCT = 95


def _pct_constants(n):
    fracs = np.linspace(MIN_PCT / 100.0, MAX_PCT / 100.0, N_PCT)
    idx_float = fracs * (n - 1)
    idx_lower = np.floor(idx_float).astype(np.int32)
    idx_upper = np.ceil(idx_float).astype(np.int32)
    w_upper = (idx_float - idx_lower).astype(np.float32)
    return idx_lower, idx_upper, w_upper


def _bitrev(r, nbits):
    return int(format(r, "0{}b".format(nbits))[::-1], 2)


def _body(x_ref, o_ref, *, n):
    dc = x_ref.shape[2]
    logn = int(np.log2(n))
    u = jax.lax.bitcast_convert_type(x_ref[0], jnp.int32)
    v = u ^ ((u >> 31) & 0x7FFFFFFF)
    row3 = jax.lax.broadcasted_iota(jnp.int32, (n // 8, 8, 1), 1)
    bit0_3 = [((row3 >> p) & 1) == 0 for p in range(3)]
    row = jax.lax.broadcasted_iota(jnp.int32, (n, 1), 0)
    mbit = [(row >> p) & 1 for p in range(logn)]
    v = v ^ -mbit[logn - 2]
    G = 3
    for k in range(1, logn + 1):
        ps = list(range(logn - k, logn))
        for p in [q for q in ps if q < 3]:
            dist = 1 << p
            w = v.reshape(n // 8, 8, dc)
            up = jnp.roll(w, -dist, axis=1)
            down = jnp.roll(w, dist, axis=1)
            partner = jnp.where(bit0_3[p], up, down)
            lo = jnp.minimum(w, partner)
            hi = jnp.maximum(w, partner)
            v = jnp.where(bit0_3[p], lo, hi).reshape(n, dc)
        big = [q for q in ps if q >= 3]
        pos = 0
        while pos < len(big):
            group = big[pos : pos + G]
            pos += len(group)
            g = len(group)
            p0 = group[0]
            dist = 1 << p0
            m = n // ((1 << g) * dist)
            w = v.reshape(m, 1 << g, dist, dc)
            sl = [w[:, q] for q in range(1 << g)]
            for e in range(g):
                step = 1 << e
                for q in range(1 << g):
                    if q & step:
                        continue
                    a, b_ = sl[q], sl[q ^ step]
                    sl[q] = jnp.minimum(a, b_)
                    sl[q ^ step] = jnp.maximum(a, b_)
            v = jnp.stack(sl, axis=1).reshape(n, dc)
        if k < logn - 1:
            v = v ^ -(mbit[logn - 1 - k] ^ mbit[logn - 2 - k])
        elif k == logn - 1:
            v = v ^ -mbit[0]

    idx_lower, idx_upper, w_upper = _pct_constants(n)
    rows = []
    for p in range(N_PCT):
        rl = _bitrev(int(idx_lower[p]), logn)
        ru = _bitrev(int(idx_upper[p]), logn)
        rows.append(v[rl : rl + 1, :])
        rows.append(v[ru : ru + 1, :])
    keys = jnp.concatenate(rows, axis=0)
    ui = keys ^ ((keys >> 31) & 0x7FFFFFFF)
    f = jax.lax.bitcast_convert_type(ui, jnp.float32)
    out = [
        f[2 * p : 2 * p + 1] * (1.0 - float(w_upper[p]))
        + f[2 * p + 1 : 2 * p + 2] * float(w_upper[p])
        for p in range(N_PCT)
    ]
    o_ref[0] = jnp.concatenate(out, axis=0)


@jax.jit
def kernel(x):
    b, n, d = x.shape
    DC = 128
    out = pl.pallas_call(
        functools.partial(_body, n=n),
        grid=(b, d // DC),
        in_specs=[pl.BlockSpec((1, n, DC), lambda i, j: (i, 0, j))],
        out_specs=pl.BlockSpec((1, N_PCT, DC), lambda i, j: (i, 0, j)),
        out_shape=jax.ShapeDtypeStruct((b, N_PCT, d), jnp.float32),
    )(x)
    return jnp.transpose(out, (0, 2, 1)).reshape(b, d * N_PCT)


# speedup vs baseline: 5.8069x; 1.0293x over previous
import functools

import jax
import jax.numpy as jnp
import numpy as np
from jax.experimental import pallas as pl

N_PCT = 10
MIN_PCT = 5
MAX_PCT = 95


def _pct_constants(n):
    fracs = np.linspace(MIN_PCT / 100.0, MAX_PCT / 100.0, N_PCT)
    idx_float = fracs * (n - 1)
    idx_lower = np.floor(idx_float).astype(np.int32)
    idx_upper = np.ceil(idx_float).astype(np.int32)
    w_upper = (idx_float - idx_lower).astype(np.float32)
    return idx_lower, idx_upper, w_upper


def _bitrev(r, nbits):
    return int(format(r, "0{}b".format(nbits))[::-1], 2)


def _body(x_ref, o_ref, *, n):
    dc = x_ref.shape[2]
    logn = int(np.log2(n))
    u = jax.lax.bitcast_convert_type(x_ref[0], jnp.int32)
    v = u ^ ((u >> 31) & 0x7FFFFFFF)
    row3 = jax.lax.broadcasted_iota(jnp.int32, (n // 8, 8, 1), 1)
    bit0_3 = [((row3 >> p) & 1) == 0 for p in range(3)]
    row = jax.lax.broadcasted_iota(jnp.int32, (n, 1), 0)
    mbit = [(row >> p) & 1 for p in range(logn)]
    v = v ^ -mbit[logn - 2]
    G = 4
    for k in range(1, logn + 1):
        ps = list(range(logn - k, logn))
        for p in [q for q in ps if q < 3]:
            dist = 1 << p
            w = v.reshape(n // 8, 8, dc)
            up = jnp.roll(w, -dist, axis=1)
            down = jnp.roll(w, dist, axis=1)
            v = jnp.where(
                bit0_3[p], jnp.minimum(w, up), jnp.maximum(w, down)
            ).reshape(n, dc)
        big = [q for q in ps if q >= 3]
        pos = 0
        while pos < len(big):
            group = big[pos : pos + G]
            pos += len(group)
            g = len(group)
            p0 = group[0]
            dist = 1 << p0
            m = n // ((1 << g) * dist)
            w = v.reshape(m, 1 << g, dist, dc)
            sl = [w[:, q] for q in range(1 << g)]
            for e in range(g):
                step = 1 << e
                for q in range(1 << g):
                    if q & step:
                        continue
                    a, b_ = sl[q], sl[q ^ step]
                    sl[q] = jnp.minimum(a, b_)
                    sl[q ^ step] = jnp.maximum(a, b_)
            v = jnp.stack(sl, axis=1).reshape(n, dc)
        if k < logn - 1:
            v = v ^ -(mbit[logn - 1 - k] ^ mbit[logn - 2 - k])
        elif k == logn - 1:
            v = v ^ -mbit[0]

    idx_lower, idx_upper, w_upper = _pct_constants(n)
    rows = []
    for p in range(N_PCT):
        rl = _bitrev(int(idx_lower[p]), logn)
        ru = _bitrev(int(idx_upper[p]), logn)
        rows.append(v[rl : rl + 1, :])
        rows.append(v[ru : ru + 1, :])
    keys = jnp.concatenate(rows, axis=0)
    ui = keys ^ ((keys >> 31) & 0x7FFFFFFF)
    f = jax.lax.bitcast_convert_type(ui, jnp.float32)
    out = [
        f[2 * p : 2 * p + 1] * (1.0 - float(w_upper[p]))
        + f[2 * p + 1 : 2 * p + 2] * float(w_upper[p])
        for p in range(N_PCT)
    ]
    o_ref[0] = jnp.concatenate(out, axis=0)


@jax.jit
def kernel(x):
    b, n, d = x.shape
    DC = 128
    out = pl.pallas_call(
        functools.partial(_body, n=n),
        grid=(b, d // DC),
        in_specs=[pl.BlockSpec((1, n, DC), lambda i, j: (i, 0, j))],
        out_specs=pl.BlockSpec((1, N_PCT, DC), lambda i, j: (i, 0, j)),
        out_shape=jax.ShapeDtypeStruct((b, N_PCT, d), jnp.float32),
    )(x)
    return jnp.transpose(out, (0, 2, 1)).reshape(b, d * N_PCT)
